# baseline (speedup 1.0000x reference)
import functools

import jax
import jax.numpy as jnp
from jax import lax
from jax.experimental import pallas as pl
from jax.experimental.pallas import tpu as pltpu
from jax.experimental.pallas import tpu_sc as plsc

N = 10000
D = 128
E = 320000
NC, NS = 2, 16
NW = NC * NS
NPAD = 10240
RB = NPAD // NS
CHUNK = 128
CH = 79
UNROLL = 8
EPT = CH * CHUNK
E_PAD = EPT * NW
TCB = 256
GRID = NPAD // TCB

_mesh = plsc.VectorSubcoreMesh(
    core_axis_name="c", subcore_axis_name="s", num_cores=NC, num_subcores=NS)


@functools.partial(
    pl.kernel,
    out_type=jax.ShapeDtypeStruct((NC, NPAD, D), jnp.float32),
    mesh=_mesh,
    scratch_types=[
        pltpu.VMEM_SHARED((NPAD, D), jnp.float32),
        pltpu.VMEM((CHUNK, D), jnp.float32),
        pltpu.VMEM((CHUNK,), jnp.int32),
    ],
)
def _sc_deg(dst_hbm, zero_hbm, ones_hbm, out_hbm, acc, ones_v, idx_v):
    c = lax.axis_index("c")
    s = lax.axis_index("s")
    wid = c * NS + s
    pltpu.sync_copy(zero_hbm, acc.at[pl.ds(s * RB, RB)])
    pltpu.sync_copy(ones_hbm, ones_v)
    plsc.subcore_barrier()

    def body(i, carry):
        base = wid * EPT + i * CHUNK
        pltpu.sync_copy(dst_hbm.at[pl.ds(base, CHUNK)], idx_v)
        pltpu.sync_copy(ones_v, acc.at[idx_v], add=True)
        return carry

    lax.fori_loop(0, CH, body, 0)
    plsc.subcore_barrier()
    pltpu.sync_copy(acc.at[pl.ds(s * RB, RB)], out_hbm.at[c, pl.ds(s * RB, RB)])


@functools.partial(
    pl.kernel,
    out_type=jax.ShapeDtypeStruct((NC, NPAD, D), jnp.float32),
    mesh=_mesh,
    scratch_types=[
        pltpu.VMEM_SHARED((NPAD, D), jnp.float32),
        pltpu.VMEM((CHUNK, D), jnp.float32),
        pltpu.VMEM((CHUNK,), jnp.int32),
        pltpu.VMEM((CHUNK,), jnp.int32),
    ],
)
def _sc_scatter(z_hbm, src_hbm, dst_hbm, zero_hbm, out_hbm,
                acc, rows_v, sidx_v, didx_v):
    c = lax.axis_index("c")
    s = lax.axis_index("s")
    wid = c * NS + s
    pltpu.sync_copy(zero_hbm, acc.at[pl.ds(s * RB, RB)])
    plsc.subcore_barrier()

    def body(i, carry):
        base = wid * EPT + i * CHUNK
        pltpu.sync_copy(src_hbm.at[pl.ds(base, CHUNK)], sidx_v)
        pltpu.sync_copy(dst_hbm.at[pl.ds(base, CHUNK)], didx_v)
        pltpu.sync_copy(z_hbm.at[sidx_v], rows_v)
        pltpu.sync_copy(rows_v, acc.at[didx_v], add=True)
        return carry

    lax.fori_loop(0, CH, body, 0)
    plsc.subcore_barrier()
    pltpu.sync_copy(acc.at[pl.ds(s * RB, RB)], out_hbm.at[c, pl.ds(s * RB, RB)])


def _dinv_block(degp):
    deg = degp[0, :, 0:1] + degp[1, :, 0:1] + 1.0
    return lax.rsqrt(deg)


def _row_mask(i):
    rows = i * TCB + lax.broadcasted_iota(jnp.int32, (TCB, 1), 0)
    return rows < N


def _tc_matmul(x_ref, w_ref, xw_ref):
    xw_ref[...] = jnp.dot(x_ref[...], w_ref[...],
                          preferred_element_type=jnp.float32)


def _tc_first(degp_ref, xw_ref, z_ref):
    i = pl.program_id(0)
    dinv = _dinv_block(degp_ref[...])
    z_ref[...] = jnp.where(_row_mask(i), xw_ref[...] * dinv, 0.0)


def _tc_mid(degp_ref, p_ref, z_ref, b_ref, w_ref, z2_ref):
    i = pl.program_id(0)
    dinv = _dinv_block(degp_ref[...])
    ssum = p_ref[0] + p_ref[1] + z_ref[...]
    h = jnp.maximum(dinv * ssum + b_ref[...], 0.0)
    h = jnp.where(_row_mask(i), h, 0.0)
    z2_ref[...] = jnp.dot(h, w_ref[...], preferred_element_type=jnp.float32) * dinv


def _tc_last(degp_ref, p_ref, z_ref, b_ref, wfc_ref, bfc_ref, y_ref):
    i = pl.program_id(0)
    dinv = _dinv_block(degp_ref[...])
    ssum = p_ref[0] + p_ref[1] + z_ref[...]
    h = jnp.maximum(dinv * ssum + b_ref[...], 0.0)
    h = jnp.where(_row_mask(i), h, 0.0)
    y_ref[...] = jnp.sum(h * wfc_ref[...], axis=1, keepdims=True) + bfc_ref[0, 0]


_degp_spec = pl.BlockSpec((NC, TCB, D), lambda i: (0, i, 0))
_rows_spec = pl.BlockSpec((TCB, D), lambda i: (i, 0))
_parts_spec = pl.BlockSpec((NC, TCB, D), lambda i: (0, i, 0))
_w_spec = pl.BlockSpec((D, D), lambda i: (0, 0))
_b_spec = pl.BlockSpec((1, D), lambda i: (0, 0))


def _tc_matmul_call(x, w):
    return pl.pallas_call(
        _tc_matmul,
        grid=(GRID,),
        in_specs=[_rows_spec, _w_spec],
        out_specs=_rows_spec,
        out_shape=jax.ShapeDtypeStruct((NPAD, D), jnp.float32),
    )(x, w)


def _tc_first_call(degp, xw):
    return pl.pallas_call(
        _tc_first,
        grid=(GRID,),
        in_specs=[_degp_spec, _rows_spec],
        out_specs=_rows_spec,
        out_shape=jax.ShapeDtypeStruct((NPAD, D), jnp.float32),
    )(degp, xw)


def _tc_mid_call(degp, parts, z, b2d, w):
    return pl.pallas_call(
        _tc_mid,
        grid=(GRID,),
        in_specs=[_degp_spec, _parts_spec, _rows_spec, _b_spec, _w_spec],
        out_specs=_rows_spec,
        out_shape=jax.ShapeDtypeStruct((NPAD, D), jnp.float32),
    )(degp, parts, z, b2d, w)


def _tc_last_call(degp, parts, z, b2d, wfc_row, bfc2d):
    return pl.pallas_call(
        _tc_last,
        grid=(GRID,),
        in_specs=[_degp_spec, _parts_spec, _rows_spec, _b_spec, _b_spec,
                  pl.BlockSpec((1, 1), lambda i: (0, 0))],
        out_specs=pl.BlockSpec((TCB, 1), lambda i: (i, 0)),
        out_shape=jax.ShapeDtypeStruct((NPAD, 1), jnp.float32),
    )(degp, parts, z, b2d, wfc_row, bfc2d)


def kernel(x, edge_index, W1, b1, W2, b2, Wfc, bfc):
    ei = edge_index.astype(jnp.int32)
    pad = jnp.full((E_PAD - E,), N, jnp.int32)
    srcp = jnp.concatenate([ei[0], pad])
    dstp = jnp.concatenate([ei[1], pad])

    zeroD = jnp.zeros((RB, D), jnp.float32)
    onesD = jnp.ones((CHUNK, D), jnp.float32)

    xw1 = _tc_matmul_call(x, W1)
    degp = _sc_deg(dstp, zeroD, onesD)

    z1 = _tc_first_call(degp, xw1)
    p1 = _sc_scatter(z1, srcp, dstp, zeroD)
    z2 = _tc_mid_call(degp, p1, z1, b1.reshape(1, D), W2)
    p2 = _sc_scatter(z2, srcp, dstp, zeroD)
    y = _tc_last_call(degp, p2, z2, b2.reshape(1, D),
                      Wfc.reshape(1, D), bfc.reshape(1, 1))
    return y[:N]


# speedup vs baseline: 1.0444x; 1.0444x over previous
import functools

import jax
import jax.numpy as jnp
from jax import lax
from jax.experimental import pallas as pl
from jax.experimental.pallas import tpu as pltpu
from jax.experimental.pallas import tpu_sc as plsc

N = 10000
D = 128
E = 320000
NC, NS = 2, 16
NW = NC * NS
NPAD = 10240
RB = NPAD // NS
CHUNK = 128
CH = 79
UNROLL = 8
EPT = CH * CHUNK
E_PAD = EPT * NW
TCB = 256
GRID = NPAD // TCB

_mesh = plsc.VectorSubcoreMesh(
    core_axis_name="c", subcore_axis_name="s", num_cores=NC, num_subcores=NS)


@functools.partial(
    pl.kernel,
    out_type=jax.ShapeDtypeStruct((NC, NPAD, D), jnp.float32),
    mesh=_mesh,
    scratch_types=[
        pltpu.VMEM_SHARED((NPAD, D), jnp.float32),
        pltpu.VMEM((CHUNK, D), jnp.float32),
        pltpu.VMEM((CHUNK,), jnp.int32),
        pltpu.VMEM((CHUNK,), jnp.int32),
        pltpu.SemaphoreType.DMA,
    ],
)
def _sc_deg(dst_hbm, zero_hbm, ones_hbm, out_hbm, acc, ones_v, idx_a, idx_b,
            sem):
    c = lax.axis_index("c")
    s = lax.axis_index("s")
    wid = c * NS + s
    pltpu.sync_copy(zero_hbm, acc.at[pl.ds(s * RB, RB)])
    pltpu.sync_copy(ones_hbm, ones_v)
    plsc.subcore_barrier()

    def body(g, carry):
        base = wid * EPT + g * (2 * CHUNK)
        pltpu.sync_copy(dst_hbm.at[pl.ds(base, CHUNK)], idx_a)
        d1 = pltpu.async_copy(ones_v, acc.at[idx_a], sem, add=True)
        pltpu.sync_copy(dst_hbm.at[pl.ds(base + CHUNK, CHUNK)], idx_b)
        d2 = pltpu.async_copy(ones_v, acc.at[idx_b], sem, add=True)
        d1.wait()
        d2.wait()
        return carry

    lax.fori_loop(0, CH // 2, body, 0)
    base = wid * EPT + (CH - 1) * CHUNK
    pltpu.sync_copy(dst_hbm.at[pl.ds(base, CHUNK)], idx_a)
    pltpu.sync_copy(ones_v, acc.at[idx_a], add=True)
    plsc.subcore_barrier()
    pltpu.sync_copy(acc.at[pl.ds(s * RB, RB)], out_hbm.at[c, pl.ds(s * RB, RB)])


@functools.partial(
    pl.kernel,
    out_type=jax.ShapeDtypeStruct((NC, NPAD, D), jnp.float32),
    mesh=_mesh,
    scratch_types=[
        pltpu.VMEM_SHARED((NPAD, D), jnp.float32),
        pltpu.VMEM((CHUNK, D), jnp.float32),
        pltpu.VMEM((CHUNK,), jnp.int32),
        pltpu.VMEM((CHUNK,), jnp.int32),
    ],
)
def _sc_scatter(z_hbm, src_hbm, dst_hbm, zero_hbm, out_hbm,
                acc, rows_v, sidx_v, didx_v):
    c = lax.axis_index("c")
    s = lax.axis_index("s")
    wid = c * NS + s
    pltpu.sync_copy(zero_hbm, acc.at[pl.ds(s * RB, RB)])
    plsc.subcore_barrier()

    def body(i, carry):
        base = wid * EPT + i * CHUNK
        pltpu.sync_copy(src_hbm.at[pl.ds(base, CHUNK)], sidx_v)
        pltpu.sync_copy(dst_hbm.at[pl.ds(base, CHUNK)], didx_v)
        pltpu.sync_copy(z_hbm.at[sidx_v], rows_v)
        pltpu.sync_copy(rows_v, acc.at[didx_v], add=True)
        return carry

    lax.fori_loop(0, CH, body, 0)
    plsc.subcore_barrier()
    pltpu.sync_copy(acc.at[pl.ds(s * RB, RB)], out_hbm.at[c, pl.ds(s * RB, RB)])


def _dinv_block(degp):
    deg = degp[0, :, 0:1] + degp[1, :, 0:1] + 1.0
    return lax.rsqrt(deg)


def _row_mask(i):
    rows = i * TCB + lax.broadcasted_iota(jnp.int32, (TCB, 1), 0)
    return rows < N


def _tc_first(degp_ref, x_ref, w_ref, z_ref):
    i = pl.program_id(0)
    dinv = _dinv_block(degp_ref[...])
    xw = jnp.dot(x_ref[...], w_ref[...], preferred_element_type=jnp.float32)
    z_ref[...] = jnp.where(_row_mask(i), xw * dinv, 0.0)


def _tc_mid(degp_ref, p_ref, z_ref, b_ref, w_ref, z2_ref):
    i = pl.program_id(0)
    dinv = _dinv_block(degp_ref[...])
    ssum = p_ref[0] + p_ref[1] + z_ref[...]
    h = jnp.maximum(dinv * ssum + b_ref[...], 0.0)
    h = jnp.where(_row_mask(i), h, 0.0)
    z2_ref[...] = jnp.dot(h, w_ref[...], preferred_element_type=jnp.float32) * dinv


def _tc_last(degp_ref, p_ref, z_ref, b_ref, wfc_ref, bfc_ref, y_ref):
    i = pl.program_id(0)
    dinv = _dinv_block(degp_ref[...])
    ssum = p_ref[0] + p_ref[1] + z_ref[...]
    h = jnp.maximum(dinv * ssum + b_ref[...], 0.0)
    h = jnp.where(_row_mask(i), h, 0.0)
    y_ref[...] = jnp.sum(h * wfc_ref[...], axis=1, keepdims=True) + bfc_ref[0, 0]


_degp_spec = pl.BlockSpec((NC, TCB, D), lambda i: (0, i, 0))
_rows_spec = pl.BlockSpec((TCB, D), lambda i: (i, 0))
_parts_spec = pl.BlockSpec((NC, TCB, D), lambda i: (0, i, 0))
_w_spec = pl.BlockSpec((D, D), lambda i: (0, 0))
_b_spec = pl.BlockSpec((1, D), lambda i: (0, 0))


def _tc_first_call(degp, x, w):
    return pl.pallas_call(
        _tc_first,
        grid=(GRID,),
        in_specs=[_degp_spec, _rows_spec, _w_spec],
        out_specs=_rows_spec,
        out_shape=jax.ShapeDtypeStruct((NPAD, D), jnp.float32),
    )(degp, x, w)


def _tc_mid_call(degp, parts, z, b2d, w):
    return pl.pallas_call(
        _tc_mid,
        grid=(GRID,),
        in_specs=[_degp_spec, _parts_spec, _rows_spec, _b_spec, _w_spec],
        out_specs=_rows_spec,
        out_shape=jax.ShapeDtypeStruct((NPAD, D), jnp.float32),
    )(degp, parts, z, b2d, w)


def _tc_last_call(degp, parts, z, b2d, wfc_row, bfc2d):
    return pl.pallas_call(
        _tc_last,
        grid=(GRID,),
        in_specs=[_degp_spec, _parts_spec, _rows_spec, _b_spec, _b_spec,
                  pl.BlockSpec((1, 1), lambda i: (0, 0))],
        out_specs=pl.BlockSpec((TCB, 1), lambda i: (i, 0)),
        out_shape=jax.ShapeDtypeStruct((NPAD, 1), jnp.float32),
    )(degp, parts, z, b2d, wfc_row, bfc2d)


def kernel(x, edge_index, W1, b1, W2, b2, Wfc, bfc):
    ei = edge_index.astype(jnp.int32)
    pad = jnp.full((E_PAD - E,), N, jnp.int32)
    srcp = jnp.concatenate([ei[0], pad])
    dstp = jnp.concatenate([ei[1], pad])

    zeroD = jnp.zeros((RB, D), jnp.float32)
    onesD = jnp.ones((CHUNK, D), jnp.float32)

    degp = _sc_deg(dstp, zeroD, onesD)

    z1 = _tc_first_call(degp, x, W1)
    p1 = _sc_scatter(z1, srcp, dstp, zeroD)
    z2 = _tc_mid_call(degp, p1, z1, b1.reshape(1, D), W2)
    p2 = _sc_scatter(z2, srcp, dstp, zeroD)
    y = _tc_last_call(degp, p2, z2, b2.reshape(1, D),
                      Wfc.reshape(1, D), bfc.reshape(1, 1))
    return y[:N]


# speedup vs baseline: 1.0984x; 1.0517x over previous
import functools

import jax
import jax.numpy as jnp
from jax import lax
from jax.experimental import pallas as pl
from jax.experimental.pallas import tpu as pltpu
from jax.experimental.pallas import tpu_sc as plsc

N = 10000
D = 128
E = 320000
NC, NS = 2, 16
NW = NC * NS
NPAD = 10240
RB = NPAD // NS
CHUNK = 128
CH = 79
UNROLL = 8
EPT = CH * CHUNK
E_PAD = EPT * NW
TCB = 256
GRID = NPAD // TCB

_mesh = plsc.VectorSubcoreMesh(
    core_axis_name="c", subcore_axis_name="s", num_cores=NC, num_subcores=NS)


@functools.partial(
    pl.kernel,
    out_type=jax.ShapeDtypeStruct((NC, NPAD, D), jnp.float32),
    mesh=_mesh,
    scratch_types=[
        pltpu.VMEM_SHARED((NPAD, D), jnp.float32),
        pltpu.VMEM((CHUNK, D), jnp.float32),
        pltpu.VMEM((CHUNK,), jnp.int32),
        pltpu.VMEM((CHUNK,), jnp.int32),
        pltpu.SemaphoreType.DMA,
    ],
)
def _sc_deg(dst_hbm, zero_hbm, ones_hbm, out_hbm, acc, ones_v, idx_a, idx_b,
            sem):
    c = lax.axis_index("c")
    s = lax.axis_index("s")
    wid = c * NS + s
    pltpu.sync_copy(zero_hbm, acc.at[pl.ds(s * RB, RB)])
    pltpu.sync_copy(ones_hbm, ones_v)
    plsc.subcore_barrier()

    def body(g, carry):
        base = wid * EPT + g * (2 * CHUNK)
        pltpu.sync_copy(dst_hbm.at[pl.ds(base, CHUNK)], idx_a)
        d1 = pltpu.async_copy(ones_v, acc.at[idx_a], sem, add=True)
        pltpu.sync_copy(dst_hbm.at[pl.ds(base + CHUNK, CHUNK)], idx_b)
        d2 = pltpu.async_copy(ones_v, acc.at[idx_b], sem, add=True)
        d1.wait()
        d2.wait()
        return carry

    lax.fori_loop(0, CH // 2, body, 0)
    base = wid * EPT + (CH - 1) * CHUNK
    pltpu.sync_copy(dst_hbm.at[pl.ds(base, CHUNK)], idx_a)
    pltpu.sync_copy(ones_v, acc.at[idx_a], add=True)
    plsc.subcore_barrier()
    pltpu.sync_copy(acc.at[pl.ds(s * RB, RB)], out_hbm.at[c, pl.ds(s * RB, RB)])


@functools.partial(
    pl.kernel,
    out_type=jax.ShapeDtypeStruct((NC, NPAD, D), jnp.float32),
    mesh=_mesh,
    scratch_types=[
        pltpu.VMEM_SHARED((NPAD, D), jnp.float32),
        pltpu.VMEM((CHUNK, D), jnp.float32),
        pltpu.VMEM((CHUNK, D), jnp.float32),
        pltpu.VMEM((CHUNK,), jnp.int32),
        pltpu.VMEM((CHUNK,), jnp.int32),
        pltpu.VMEM((CHUNK,), jnp.int32),
        pltpu.SemaphoreType.DMA,
    ],
)
def _sc_scatter(z_hbm, src_hbm, dst_hbm, zero_hbm, out_hbm,
                acc, rows_a, rows_b, sidx_v, didx_a, didx_b, sem):
    c = lax.axis_index("c")
    s = lax.axis_index("s")
    wid = c * NS + s
    pltpu.sync_copy(zero_hbm, acc.at[pl.ds(s * RB, RB)])
    plsc.subcore_barrier()

    def body(g, carry):
        base = wid * EPT + g * (2 * CHUNK)
        pltpu.sync_copy(src_hbm.at[pl.ds(base, CHUNK)], sidx_v)
        pltpu.sync_copy(dst_hbm.at[pl.ds(base, CHUNK)], didx_a)
        pltpu.sync_copy(z_hbm.at[sidx_v], rows_a)
        d1 = pltpu.async_copy(rows_a, acc.at[didx_a], sem, add=True)
        pltpu.sync_copy(src_hbm.at[pl.ds(base + CHUNK, CHUNK)], sidx_v)
        pltpu.sync_copy(dst_hbm.at[pl.ds(base + CHUNK, CHUNK)], didx_b)
        pltpu.sync_copy(z_hbm.at[sidx_v], rows_b)
        d2 = pltpu.async_copy(rows_b, acc.at[didx_b], sem, add=True)
        d1.wait()
        d2.wait()
        return carry

    lax.fori_loop(0, CH // 2, body, 0)
    base = wid * EPT + (CH - 1) * CHUNK
    pltpu.sync_copy(src_hbm.at[pl.ds(base, CHUNK)], sidx_v)
    pltpu.sync_copy(dst_hbm.at[pl.ds(base, CHUNK)], didx_a)
    pltpu.sync_copy(z_hbm.at[sidx_v], rows_a)
    pltpu.sync_copy(rows_a, acc.at[didx_a], add=True)
    plsc.subcore_barrier()
    pltpu.sync_copy(acc.at[pl.ds(s * RB, RB)], out_hbm.at[c, pl.ds(s * RB, RB)])


def _dinv_block(degp):
    deg = degp[0, :, 0:1] + degp[1, :, 0:1] + 1.0
    return lax.rsqrt(deg)


def _row_mask(i):
    rows = i * TCB + lax.broadcasted_iota(jnp.int32, (TCB, 1), 0)
    return rows < N


def _tc_first(degp_ref, x_ref, w_ref, z_ref):
    i = pl.program_id(0)
    dinv = _dinv_block(degp_ref[...])
    xw = jnp.dot(x_ref[...], w_ref[...], preferred_element_type=jnp.float32)
    z_ref[...] = jnp.where(_row_mask(i), xw * dinv, 0.0)


def _tc_mid(degp_ref, p_ref, z_ref, b_ref, w_ref, z2_ref):
    i = pl.program_id(0)
    dinv = _dinv_block(degp_ref[...])
    ssum = p_ref[0] + p_ref[1] + z_ref[...]
    h = jnp.maximum(dinv * ssum + b_ref[...], 0.0)
    h = jnp.where(_row_mask(i), h, 0.0)
    z2_ref[...] = jnp.dot(h, w_ref[...], preferred_element_type=jnp.float32) * dinv


def _tc_last(degp_ref, p_ref, z_ref, b_ref, wfc_ref, bfc_ref, y_ref):
    i = pl.program_id(0)
    dinv = _dinv_block(degp_ref[...])
    ssum = p_ref[0] + p_ref[1] + z_ref[...]
    h = jnp.maximum(dinv * ssum + b_ref[...], 0.0)
    h = jnp.where(_row_mask(i), h, 0.0)
    y_ref[...] = jnp.sum(h * wfc_ref[...], axis=1, keepdims=True) + bfc_ref[0, 0]


_degp_spec = pl.BlockSpec((NC, TCB, D), lambda i: (0, i, 0))
_rows_spec = pl.BlockSpec((TCB, D), lambda i: (i, 0))
_parts_spec = pl.BlockSpec((NC, TCB, D), lambda i: (0, i, 0))
_w_spec = pl.BlockSpec((D, D), lambda i: (0, 0))
_b_spec = pl.BlockSpec((1, D), lambda i: (0, 0))


def _tc_first_call(degp, x, w):
    return pl.pallas_call(
        _tc_first,
        grid=(GRID,),
        in_specs=[_degp_spec, _rows_spec, _w_spec],
        out_specs=_rows_spec,
        out_shape=jax.ShapeDtypeStruct((NPAD, D), jnp.float32),
    )(degp, x, w)


def _tc_mid_call(degp, parts, z, b2d, w):
    return pl.pallas_call(
        _tc_mid,
        grid=(GRID,),
        in_specs=[_degp_spec, _parts_spec, _rows_spec, _b_spec, _w_spec],
        out_specs=_rows_spec,
        out_shape=jax.ShapeDtypeStruct((NPAD, D), jnp.float32),
    )(degp, parts, z, b2d, w)


def _tc_last_call(degp, parts, z, b2d, wfc_row, bfc2d):
    return pl.pallas_call(
        _tc_last,
        grid=(GRID,),
        in_specs=[_degp_spec, _parts_spec, _rows_spec, _b_spec, _b_spec,
                  pl.BlockSpec((1, 1), lambda i: (0, 0))],
        out_specs=pl.BlockSpec((TCB, 1), lambda i: (i, 0)),
        out_shape=jax.ShapeDtypeStruct((NPAD, 1), jnp.float32),
    )(degp, parts, z, b2d, wfc_row, bfc2d)


def kernel(x, edge_index, W1, b1, W2, b2, Wfc, bfc):
    ei = edge_index.astype(jnp.int32)
    pad = jnp.full((E_PAD - E,), N, jnp.int32)
    srcp = jnp.concatenate([ei[0], pad])
    dstp = jnp.concatenate([ei[1], pad])

    zeroD = jnp.zeros((RB, D), jnp.float32)
    onesD = jnp.ones((CHUNK, D), jnp.float32)

    degp = _sc_deg(dstp, zeroD, onesD)

    z1 = _tc_first_call(degp, x, W1)
    p1 = _sc_scatter(z1, srcp, dstp, zeroD)
    z2 = _tc_mid_call(degp, p1, z1, b1.reshape(1, D), W2)
    p2 = _sc_scatter(z2, srcp, dstp, zeroD)
    y = _tc_last_call(degp, p2, z2, b2.reshape(1, D),
                      Wfc.reshape(1, D), bfc.reshape(1, 1))
    return y[:N]


# speedup vs baseline: 1.1752x; 1.0700x over previous
import functools

import jax
import jax.numpy as jnp
from jax import lax
from jax.experimental import pallas as pl
from jax.experimental.pallas import tpu as pltpu
from jax.experimental.pallas import tpu_sc as plsc

N = 10000
D = 128
E = 320000
NC, NS = 2, 16
NW = NC * NS
NPAD = 10240
RB = NPAD // NS
CHUNK = 128
CH = 79
UNROLL = 8
EPT = CH * CHUNK
E_PAD = EPT * NW
TCB = 256
GRID = NPAD // TCB

_mesh = plsc.VectorSubcoreMesh(
    core_axis_name="c", subcore_axis_name="s", num_cores=NC, num_subcores=NS)


@functools.partial(
    pl.kernel,
    out_type=jax.ShapeDtypeStruct((NC, NPAD, D), jnp.float32),
    mesh=_mesh,
    scratch_types=[
        pltpu.VMEM_SHARED((NPAD, D), jnp.float32),
        pltpu.VMEM((CHUNK, D), jnp.float32),
        pltpu.VMEM((CHUNK,), jnp.int32),
        pltpu.VMEM((CHUNK,), jnp.int32),
        pltpu.SemaphoreType.DMA,
    ],
)
def _sc_deg(dst_hbm, zero_hbm, ones_hbm, out_hbm, acc, ones_v, idx_a, idx_b,
            sem):
    c = lax.axis_index("c")
    s = lax.axis_index("s")
    wid = c * NS + s
    pltpu.sync_copy(zero_hbm, acc.at[pl.ds(s * RB, RB)])
    pltpu.sync_copy(ones_hbm, ones_v)
    plsc.subcore_barrier()

    def body(g, carry):
        base = wid * EPT + g * (2 * CHUNK)
        pltpu.sync_copy(dst_hbm.at[pl.ds(base, CHUNK)], idx_a)
        d1 = pltpu.async_copy(ones_v, acc.at[idx_a], sem, add=True)
        pltpu.sync_copy(dst_hbm.at[pl.ds(base + CHUNK, CHUNK)], idx_b)
        d2 = pltpu.async_copy(ones_v, acc.at[idx_b], sem, add=True)
        d1.wait()
        d2.wait()
        return carry

    lax.fori_loop(0, CH // 2, body, 0)
    base = wid * EPT + (CH - 1) * CHUNK
    pltpu.sync_copy(dst_hbm.at[pl.ds(base, CHUNK)], idx_a)
    pltpu.sync_copy(ones_v, acc.at[idx_a], add=True)
    plsc.subcore_barrier()
    pltpu.sync_copy(acc.at[pl.ds(s * RB, RB)], out_hbm.at[c, pl.ds(s * RB, RB)])


@functools.partial(
    pl.kernel,
    out_type=jax.ShapeDtypeStruct((NC, NPAD, D), jnp.float32),
    mesh=_mesh,
    scratch_types=[
        pltpu.VMEM_SHARED((NPAD, D), jnp.float32),
        pltpu.VMEM((CHUNK, D), jnp.float32),
        pltpu.VMEM((CHUNK, D), jnp.float32),
        pltpu.VMEM((CHUNK,), jnp.int32),
        pltpu.VMEM((CHUNK,), jnp.int32),
        pltpu.VMEM((CHUNK,), jnp.int32),
        pltpu.VMEM((CHUNK,), jnp.int32),
        pltpu.SemaphoreType.DMA,
        pltpu.SemaphoreType.DMA,
    ],
)
def _sc_scatter(z_hbm, src_hbm, dst_hbm, zero_hbm, out_hbm,
                acc, rows_a, rows_b, sidx_a, sidx_b, didx_a, didx_b,
                gsem, ssem):
    c = lax.axis_index("c")
    s = lax.axis_index("s")
    wid = c * NS + s
    pltpu.sync_copy(zero_hbm, acc.at[pl.ds(s * RB, RB)])
    plsc.subcore_barrier()

    def body(g, carry):
        base = wid * EPT + g * (2 * CHUNK)
        pltpu.sync_copy(src_hbm.at[pl.ds(base, CHUNK)], sidx_a)
        pltpu.sync_copy(dst_hbm.at[pl.ds(base, CHUNK)], didx_a)
        ga = pltpu.async_copy(z_hbm.at[sidx_a], rows_a, gsem)
        pltpu.sync_copy(src_hbm.at[pl.ds(base + CHUNK, CHUNK)], sidx_b)
        pltpu.sync_copy(dst_hbm.at[pl.ds(base + CHUNK, CHUNK)], didx_b)
        ga.wait()
        gb = pltpu.async_copy(z_hbm.at[sidx_b], rows_b, gsem)
        d1 = pltpu.async_copy(rows_a, acc.at[didx_a], ssem, add=True)
        gb.wait()
        d2 = pltpu.async_copy(rows_b, acc.at[didx_b], ssem, add=True)
        d1.wait()
        d2.wait()
        return carry

    lax.fori_loop(0, CH // 2, body, 0)
    base = wid * EPT + (CH - 1) * CHUNK
    pltpu.sync_copy(src_hbm.at[pl.ds(base, CHUNK)], sidx_a)
    pltpu.sync_copy(dst_hbm.at[pl.ds(base, CHUNK)], didx_a)
    pltpu.sync_copy(z_hbm.at[sidx_a], rows_a)
    pltpu.sync_copy(rows_a, acc.at[didx_a], add=True)
    plsc.subcore_barrier()
    pltpu.sync_copy(acc.at[pl.ds(s * RB, RB)], out_hbm.at[c, pl.ds(s * RB, RB)])


def _dinv_block(degp):
    deg = degp[0, :, 0:1] + degp[1, :, 0:1] + 1.0
    return lax.rsqrt(deg)


def _row_mask(i):
    rows = i * TCB + lax.broadcasted_iota(jnp.int32, (TCB, 1), 0)
    return rows < N


def _tc_first(degp_ref, x_ref, w_ref, z_ref):
    i = pl.program_id(0)
    dinv = _dinv_block(degp_ref[...])
    xw = jnp.dot(x_ref[...], w_ref[...], preferred_element_type=jnp.float32)
    z_ref[...] = jnp.where(_row_mask(i), xw * dinv, 0.0)


def _tc_mid(degp_ref, p_ref, z_ref, b_ref, w_ref, z2_ref):
    i = pl.program_id(0)
    dinv = _dinv_block(degp_ref[...])
    ssum = p_ref[0] + p_ref[1] + z_ref[...]
    h = jnp.maximum(dinv * ssum + b_ref[...], 0.0)
    h = jnp.where(_row_mask(i), h, 0.0)
    z2_ref[...] = jnp.dot(h, w_ref[...], preferred_element_type=jnp.float32) * dinv


def _tc_last(degp_ref, p_ref, z_ref, b_ref, wfc_ref, bfc_ref, y_ref):
    i = pl.program_id(0)
    dinv = _dinv_block(degp_ref[...])
    ssum = p_ref[0] + p_ref[1] + z_ref[...]
    h = jnp.maximum(dinv * ssum + b_ref[...], 0.0)
    h = jnp.where(_row_mask(i), h, 0.0)
    y_ref[...] = jnp.sum(h * wfc_ref[...], axis=1, keepdims=True) + bfc_ref[0, 0]


_degp_spec = pl.BlockSpec((NC, TCB, D), lambda i: (0, i, 0))
_rows_spec = pl.BlockSpec((TCB, D), lambda i: (i, 0))
_parts_spec = pl.BlockSpec((NC, TCB, D), lambda i: (0, i, 0))
_w_spec = pl.BlockSpec((D, D), lambda i: (0, 0))
_b_spec = pl.BlockSpec((1, D), lambda i: (0, 0))


def _tc_first_call(degp, x, w):
    return pl.pallas_call(
        _tc_first,
        grid=(GRID,),
        in_specs=[_degp_spec, _rows_spec, _w_spec],
        out_specs=_rows_spec,
        out_shape=jax.ShapeDtypeStruct((NPAD, D), jnp.float32),
    )(degp, x, w)


def _tc_mid_call(degp, parts, z, b2d, w):
    return pl.pallas_call(
        _tc_mid,
        grid=(GRID,),
        in_specs=[_degp_spec, _parts_spec, _rows_spec, _b_spec, _w_spec],
        out_specs=_rows_spec,
        out_shape=jax.ShapeDtypeStruct((NPAD, D), jnp.float32),
    )(degp, parts, z, b2d, w)


def _tc_last_call(degp, parts, z, b2d, wfc_row, bfc2d):
    return pl.pallas_call(
        _tc_last,
        grid=(GRID,),
        in_specs=[_degp_spec, _parts_spec, _rows_spec, _b_spec, _b_spec,
                  pl.BlockSpec((1, 1), lambda i: (0, 0))],
        out_specs=pl.BlockSpec((TCB, 1), lambda i: (i, 0)),
        out_shape=jax.ShapeDtypeStruct((NPAD, 1), jnp.float32),
    )(degp, parts, z, b2d, wfc_row, bfc2d)


def kernel(x, edge_index, W1, b1, W2, b2, Wfc, bfc):
    ei = edge_index.astype(jnp.int32)
    pad = jnp.full((E_PAD - E,), N, jnp.int32)
    srcp = jnp.concatenate([ei[0], pad])
    dstp = jnp.concatenate([ei[1], pad])

    zeroD = jnp.zeros((RB, D), jnp.float32)
    onesD = jnp.ones((CHUNK, D), jnp.float32)

    degp = _sc_deg(dstp, zeroD, onesD)

    z1 = _tc_first_call(degp, x, W1)
    p1 = _sc_scatter(z1, srcp, dstp, zeroD)
    z2 = _tc_mid_call(degp, p1, z1, b1.reshape(1, D), W2)
    p2 = _sc_scatter(z2, srcp, dstp, zeroD)
    y = _tc_last_call(degp, p2, z2, b2.reshape(1, D),
                      Wfc.reshape(1, D), bfc.reshape(1, 1))
    return y[:N]


# speedup vs baseline: 1.1845x; 1.0079x over previous
import functools

import jax
import jax.numpy as jnp
from jax import lax
from jax.experimental import pallas as pl
from jax.experimental.pallas import tpu as pltpu
from jax.experimental.pallas import tpu_sc as plsc

N = 10000
D = 128
E = 320000
NC, NS = 2, 16
NW = NC * NS
NPAD = 10240
RB = NPAD // NS
CHUNK = 128
CH = 79
UNROLL = 8
EPT = CH * CHUNK
E_PAD = EPT * NW
TCB = 256
GRID = NPAD // TCB

_mesh = plsc.VectorSubcoreMesh(
    core_axis_name="c", subcore_axis_name="s", num_cores=NC, num_subcores=NS)


@functools.partial(
    pl.kernel,
    out_type=jax.ShapeDtypeStruct((NC, NPAD, D), jnp.float32),
    mesh=_mesh,
    scratch_types=[
        pltpu.VMEM_SHARED((NPAD, D), jnp.float32),
        pltpu.VMEM((CHUNK, D), jnp.float32),
        pltpu.VMEM((CHUNK,), jnp.int32),
        pltpu.VMEM((CHUNK,), jnp.int32),
        pltpu.VMEM((CHUNK,), jnp.int32),
        pltpu.VMEM((CHUNK,), jnp.int32),
        pltpu.SemaphoreType.DMA,
    ],
)
def _sc_deg(dst_hbm, zero_hbm, ones_hbm, out_hbm, acc, ones_v, idx_a, idx_b,
            idx_c, idx_d, sem):
    c = lax.axis_index("c")
    s = lax.axis_index("s")
    wid = c * NS + s
    pltpu.sync_copy(zero_hbm, acc.at[pl.ds(s * RB, RB)])
    pltpu.sync_copy(ones_hbm, ones_v)
    plsc.subcore_barrier()

    idxs = (idx_a, idx_b, idx_c, idx_d)

    def body(g, carry):
        base = wid * EPT + g * (4 * CHUNK)
        descs = []
        for j in range(4):
            pltpu.sync_copy(dst_hbm.at[pl.ds(base + j * CHUNK, CHUNK)],
                            idxs[j])
            descs.append(
                pltpu.async_copy(ones_v, acc.at[idxs[j]], sem, add=True))
        for d in descs:
            d.wait()
        return carry

    lax.fori_loop(0, CH // 4, body, 0)
    for k in range((CH // 4) * 4, CH):
        base = wid * EPT + k * CHUNK
        pltpu.sync_copy(dst_hbm.at[pl.ds(base, CHUNK)], idx_a)
        pltpu.sync_copy(ones_v, acc.at[idx_a], add=True)
    plsc.subcore_barrier()
    pltpu.sync_copy(acc.at[pl.ds(s * RB, RB)], out_hbm.at[c, pl.ds(s * RB, RB)])


@functools.partial(
    pl.kernel,
    out_type=jax.ShapeDtypeStruct((NC, NPAD, D), jnp.float32),
    mesh=_mesh,
    scratch_types=[
        pltpu.VMEM_SHARED((NPAD, D), jnp.float32),
        pltpu.VMEM((CHUNK, D), jnp.float32),
        pltpu.VMEM((CHUNK, D), jnp.float32),
        pltpu.VMEM((CHUNK,), jnp.int32),
        pltpu.VMEM((CHUNK,), jnp.int32),
        pltpu.VMEM((CHUNK,), jnp.int32),
        pltpu.VMEM((CHUNK,), jnp.int32),
        pltpu.SemaphoreType.DMA,
        pltpu.SemaphoreType.DMA,
    ],
)
def _sc_scatter(z_hbm, src_hbm, dst_hbm, zero_hbm, out_hbm,
                acc, rows_a, rows_b, sidx_a, sidx_b, didx_a, didx_b,
                gsem, ssem):
    c = lax.axis_index("c")
    s = lax.axis_index("s")
    wid = c * NS + s
    pltpu.sync_copy(zero_hbm, acc.at[pl.ds(s * RB, RB)])
    plsc.subcore_barrier()

    def body(g, carry):
        base = wid * EPT + g * (2 * CHUNK)
        pltpu.sync_copy(src_hbm.at[pl.ds(base, CHUNK)], sidx_a)
        pltpu.sync_copy(dst_hbm.at[pl.ds(base, CHUNK)], didx_a)
        ga = pltpu.async_copy(z_hbm.at[sidx_a], rows_a, gsem)
        pltpu.sync_copy(src_hbm.at[pl.ds(base + CHUNK, CHUNK)], sidx_b)
        pltpu.sync_copy(dst_hbm.at[pl.ds(base + CHUNK, CHUNK)], didx_b)
        ga.wait()
        gb = pltpu.async_copy(z_hbm.at[sidx_b], rows_b, gsem)
        d1 = pltpu.async_copy(rows_a, acc.at[didx_a], ssem, add=True)
        gb.wait()
        d2 = pltpu.async_copy(rows_b, acc.at[didx_b], ssem, add=True)
        d1.wait()
        d2.wait()
        return carry

    lax.fori_loop(0, CH // 2, body, 0)
    base = wid * EPT + (CH - 1) * CHUNK
    pltpu.sync_copy(src_hbm.at[pl.ds(base, CHUNK)], sidx_a)
    pltpu.sync_copy(dst_hbm.at[pl.ds(base, CHUNK)], didx_a)
    pltpu.sync_copy(z_hbm.at[sidx_a], rows_a)
    pltpu.sync_copy(rows_a, acc.at[didx_a], add=True)
    plsc.subcore_barrier()
    pltpu.sync_copy(acc.at[pl.ds(s * RB, RB)], out_hbm.at[c, pl.ds(s * RB, RB)])


def _dinv_block(degp):
    deg = degp[0, :, 0:1] + degp[1, :, 0:1] + 1.0
    return lax.rsqrt(deg)


def _row_mask(i):
    rows = i * TCB + lax.broadcasted_iota(jnp.int32, (TCB, 1), 0)
    return rows < N


def _tc_first(degp_ref, x_ref, w_ref, z_ref):
    i = pl.program_id(0)
    dinv = _dinv_block(degp_ref[...])
    xw = jnp.dot(x_ref[...], w_ref[...], preferred_element_type=jnp.float32)
    z_ref[...] = jnp.where(_row_mask(i), xw * dinv, 0.0)


def _tc_mid(degp_ref, p_ref, z_ref, b_ref, w_ref, z2_ref):
    i = pl.program_id(0)
    dinv = _dinv_block(degp_ref[...])
    ssum = p_ref[0] + p_ref[1] + z_ref[...]
    h = jnp.maximum(dinv * ssum + b_ref[...], 0.0)
    h = jnp.where(_row_mask(i), h, 0.0)
    z2_ref[...] = jnp.dot(h, w_ref[...], preferred_element_type=jnp.float32) * dinv


def _tc_last(degp_ref, p_ref, z_ref, b_ref, wfc_ref, bfc_ref, y_ref):
    i = pl.program_id(0)
    dinv = _dinv_block(degp_ref[...])
    ssum = p_ref[0] + p_ref[1] + z_ref[...]
    h = jnp.maximum(dinv * ssum + b_ref[...], 0.0)
    h = jnp.where(_row_mask(i), h, 0.0)
    y_ref[...] = jnp.sum(h * wfc_ref[...], axis=1, keepdims=True) + bfc_ref[0, 0]


_degp_spec = pl.BlockSpec((NC, TCB, D), lambda i: (0, i, 0))
_rows_spec = pl.BlockSpec((TCB, D), lambda i: (i, 0))
_parts_spec = pl.BlockSpec((NC, TCB, D), lambda i: (0, i, 0))
_w_spec = pl.BlockSpec((D, D), lambda i: (0, 0))
_b_spec = pl.BlockSpec((1, D), lambda i: (0, 0))


def _tc_first_call(degp, x, w):
    return pl.pallas_call(
        _tc_first,
        grid=(GRID,),
        in_specs=[_degp_spec, _rows_spec, _w_spec],
        out_specs=_rows_spec,
        out_shape=jax.ShapeDtypeStruct((NPAD, D), jnp.float32),
    )(degp, x, w)


def _tc_mid_call(degp, parts, z, b2d, w):
    return pl.pallas_call(
        _tc_mid,
        grid=(GRID,),
        in_specs=[_degp_spec, _parts_spec, _rows_spec, _b_spec, _w_spec],
        out_specs=_rows_spec,
        out_shape=jax.ShapeDtypeStruct((NPAD, D), jnp.float32),
    )(degp, parts, z, b2d, w)


def _tc_last_call(degp, parts, z, b2d, wfc_row, bfc2d):
    return pl.pallas_call(
        _tc_last,
        grid=(GRID,),
        in_specs=[_degp_spec, _parts_spec, _rows_spec, _b_spec, _b_spec,
                  pl.BlockSpec((1, 1), lambda i: (0, 0))],
        out_specs=pl.BlockSpec((TCB, 1), lambda i: (i, 0)),
        out_shape=jax.ShapeDtypeStruct((NPAD, 1), jnp.float32),
    )(degp, parts, z, b2d, wfc_row, bfc2d)


def kernel(x, edge_index, W1, b1, W2, b2, Wfc, bfc):
    ei = edge_index.astype(jnp.int32)
    pad = jnp.full((E_PAD - E,), N, jnp.int32)
    srcp = jnp.concatenate([ei[0], pad])
    dstp = jnp.concatenate([ei[1], pad])

    zeroD = jnp.zeros((RB, D), jnp.float32)
    onesD = jnp.ones((CHUNK, D), jnp.float32)

    degp = _sc_deg(dstp, zeroD, onesD)

    z1 = _tc_first_call(degp, x, W1)
    p1 = _sc_scatter(z1, srcp, dstp, zeroD)
    z2 = _tc_mid_call(degp, p1, z1, b1.reshape(1, D), W2)
    p2 = _sc_scatter(z2, srcp, dstp, zeroD)
    y = _tc_last_call(degp, p2, z2, b2.reshape(1, D),
                      Wfc.reshape(1, D), bfc.reshape(1, 1))
    return y[:N]


# speedup vs baseline: 1.2537x; 1.0585x over previous
import functools

import jax
import jax.numpy as jnp
from jax import lax
from jax.experimental import pallas as pl
from jax.experimental.pallas import tpu as pltpu
from jax.experimental.pallas import tpu_sc as plsc

N = 10000
D = 128
E = 320000
NC, NS = 2, 16
NW = NC * NS
NPAD = 10240
RB = NPAD // NS
CHUNK = 128
CH = 79
UNROLL = 8
EPT = CH * CHUNK
E_PAD = EPT * NW
TCB = 256
GRID = NPAD // TCB

_mesh = plsc.VectorSubcoreMesh(
    core_axis_name="c", subcore_axis_name="s", num_cores=NC, num_subcores=NS)


@functools.partial(
    pl.kernel,
    out_type=jax.ShapeDtypeStruct((NC, NPAD, D), jnp.float32),
    mesh=_mesh,
    scratch_types=[
        pltpu.VMEM_SHARED((NPAD, D), jnp.float32),
        pltpu.VMEM((CHUNK, D), jnp.float32),
        pltpu.VMEM((CHUNK,), jnp.int32),
        pltpu.VMEM((CHUNK,), jnp.int32),
        pltpu.VMEM((CHUNK,), jnp.int32),
        pltpu.VMEM((CHUNK,), jnp.int32),
        pltpu.SemaphoreType.DMA,
    ],
)
def _sc_deg(dst_hbm, zero_hbm, ones_hbm, out_hbm, acc, ones_v, idx_a, idx_b,
            idx_c, idx_d, sem):
    c = lax.axis_index("c")
    s = lax.axis_index("s")
    wid = c * NS + s
    pltpu.sync_copy(zero_hbm, acc.at[pl.ds(s * RB, RB)])
    pltpu.sync_copy(ones_hbm, ones_v)
    plsc.subcore_barrier()

    idxs = (idx_a, idx_b, idx_c, idx_d)

    def body(g, carry):
        base = wid * EPT + g * (4 * CHUNK)
        descs = []
        for j in range(4):
            pltpu.sync_copy(dst_hbm.at[pl.ds(base + j * CHUNK, CHUNK)],
                            idxs[j])
            descs.append(
                pltpu.async_copy(ones_v, acc.at[idxs[j]], sem, add=True))
        for d in descs:
            d.wait()
        return carry

    lax.fori_loop(0, CH // 4, body, 0)
    for k in range((CH // 4) * 4, CH):
        base = wid * EPT + k * CHUNK
        pltpu.sync_copy(dst_hbm.at[pl.ds(base, CHUNK)], idx_a)
        pltpu.sync_copy(ones_v, acc.at[idx_a], add=True)
    plsc.subcore_barrier()
    pltpu.sync_copy(acc.at[pl.ds(s * RB, RB)], out_hbm.at[c, pl.ds(s * RB, RB)])


@functools.partial(
    pl.kernel,
    out_type=jax.ShapeDtypeStruct((NC, NPAD, D), jnp.float32),
    mesh=_mesh,
    scratch_types=[
        pltpu.VMEM_SHARED((NPAD, D), jnp.float32),
        pltpu.VMEM((CHUNK, D), jnp.float32),
        pltpu.VMEM((CHUNK, D), jnp.float32),
        pltpu.VMEM((CHUNK,), jnp.int32),
        pltpu.VMEM((CHUNK,), jnp.int32),
        pltpu.VMEM((CHUNK,), jnp.int32),
        pltpu.VMEM((CHUNK,), jnp.int32),
        pltpu.SemaphoreType.DMA,
        pltpu.SemaphoreType.DMA,
    ],
)
def _sc_scatter(z_hbm, src_hbm, dst_hbm, zero_hbm, out_hbm,
                acc, rows_a, rows_b, sidx_a, sidx_b, didx_a, didx_b,
                gsem, ssem):
    c = lax.axis_index("c")
    s = lax.axis_index("s")
    wid = c * NS + s
    pltpu.sync_copy(zero_hbm, acc.at[pl.ds(s * RB, RB)])
    plsc.subcore_barrier()

    def body(g, carry):
        base = wid * EPT + g * (2 * CHUNK)
        pltpu.sync_copy(src_hbm.at[pl.ds(base, CHUNK)], sidx_a)
        pltpu.sync_copy(dst_hbm.at[pl.ds(base, CHUNK)], didx_a)
        ga = pltpu.async_copy(z_hbm.at[sidx_a], rows_a, gsem)

        @pl.when(g > 0)
        def _drain():
            pltpu.make_async_copy(rows_b, acc.at[didx_b], ssem).wait()

        pltpu.sync_copy(src_hbm.at[pl.ds(base + CHUNK, CHUNK)], sidx_b)
        pltpu.sync_copy(dst_hbm.at[pl.ds(base + CHUNK, CHUNK)], didx_b)
        ga.wait()
        gb = pltpu.async_copy(z_hbm.at[sidx_b], rows_b, gsem)
        d1 = pltpu.async_copy(rows_a, acc.at[didx_a], ssem, add=True)
        gb.wait()
        pltpu.async_copy(rows_b, acc.at[didx_b], ssem, add=True)
        d1.wait()
        return carry

    lax.fori_loop(0, CH // 2, body, 0)
    pltpu.make_async_copy(rows_b, acc.at[didx_b], ssem).wait()
    base = wid * EPT + (CH - 1) * CHUNK
    pltpu.sync_copy(src_hbm.at[pl.ds(base, CHUNK)], sidx_a)
    pltpu.sync_copy(dst_hbm.at[pl.ds(base, CHUNK)], didx_a)
    pltpu.sync_copy(z_hbm.at[sidx_a], rows_a)
    pltpu.sync_copy(rows_a, acc.at[didx_a], add=True)
    plsc.subcore_barrier()
    pltpu.sync_copy(acc.at[pl.ds(s * RB, RB)], out_hbm.at[c, pl.ds(s * RB, RB)])


def _dinv_block(degp):
    deg = degp[0, :, 0:1] + degp[1, :, 0:1] + 1.0
    return lax.rsqrt(deg)


def _row_mask(i):
    rows = i * TCB + lax.broadcasted_iota(jnp.int32, (TCB, 1), 0)
    return rows < N


def _tc_first(degp_ref, x_ref, w_ref, z_ref):
    i = pl.program_id(0)
    dinv = _dinv_block(degp_ref[...])
    xw = jnp.dot(x_ref[...], w_ref[...], preferred_element_type=jnp.float32)
    z_ref[...] = jnp.where(_row_mask(i), xw * dinv, 0.0)


def _tc_mid(degp_ref, p_ref, z_ref, b_ref, w_ref, z2_ref):
    i = pl.program_id(0)
    dinv = _dinv_block(degp_ref[...])
    ssum = p_ref[0] + p_ref[1] + z_ref[...]
    h = jnp.maximum(dinv * ssum + b_ref[...], 0.0)
    h = jnp.where(_row_mask(i), h, 0.0)
    z2_ref[...] = jnp.dot(h, w_ref[...], preferred_element_type=jnp.float32) * dinv


def _tc_last(degp_ref, p_ref, z_ref, b_ref, wfc_ref, bfc_ref, y_ref):
    i = pl.program_id(0)
    dinv = _dinv_block(degp_ref[...])
    ssum = p_ref[0] + p_ref[1] + z_ref[...]
    h = jnp.maximum(dinv * ssum + b_ref[...], 0.0)
    h = jnp.where(_row_mask(i), h, 0.0)
    y_ref[...] = jnp.sum(h * wfc_ref[...], axis=1, keepdims=True) + bfc_ref[0, 0]


_degp_spec = pl.BlockSpec((NC, TCB, D), lambda i: (0, i, 0))
_rows_spec = pl.BlockSpec((TCB, D), lambda i: (i, 0))
_parts_spec = pl.BlockSpec((NC, TCB, D), lambda i: (0, i, 0))
_w_spec = pl.BlockSpec((D, D), lambda i: (0, 0))
_b_spec = pl.BlockSpec((1, D), lambda i: (0, 0))


def _tc_first_call(degp, x, w):
    return pl.pallas_call(
        _tc_first,
        grid=(GRID,),
        in_specs=[_degp_spec, _rows_spec, _w_spec],
        out_specs=_rows_spec,
        out_shape=jax.ShapeDtypeStruct((NPAD, D), jnp.float32),
    )(degp, x, w)


def _tc_mid_call(degp, parts, z, b2d, w):
    return pl.pallas_call(
        _tc_mid,
        grid=(GRID,),
        in_specs=[_degp_spec, _parts_spec, _rows_spec, _b_spec, _w_spec],
        out_specs=_rows_spec,
        out_shape=jax.ShapeDtypeStruct((NPAD, D), jnp.float32),
    )(degp, parts, z, b2d, w)


def _tc_last_call(degp, parts, z, b2d, wfc_row, bfc2d):
    return pl.pallas_call(
        _tc_last,
        grid=(GRID,),
        in_specs=[_degp_spec, _parts_spec, _rows_spec, _b_spec, _b_spec,
                  pl.BlockSpec((1, 1), lambda i: (0, 0))],
        out_specs=pl.BlockSpec((TCB, 1), lambda i: (i, 0)),
        out_shape=jax.ShapeDtypeStruct((NPAD, 1), jnp.float32),
    )(degp, parts, z, b2d, wfc_row, bfc2d)


def kernel(x, edge_index, W1, b1, W2, b2, Wfc, bfc):
    ei = edge_index.astype(jnp.int32)
    pad = jnp.full((E_PAD - E,), N, jnp.int32)
    srcp = jnp.concatenate([ei[0], pad])
    dstp = jnp.concatenate([ei[1], pad])

    zeroD = jnp.zeros((RB, D), jnp.float32)
    onesD = jnp.ones((CHUNK, D), jnp.float32)

    degp = _sc_deg(dstp, zeroD, onesD)

    z1 = _tc_first_call(degp, x, W1)
    p1 = _sc_scatter(z1, srcp, dstp, zeroD)
    z2 = _tc_mid_call(degp, p1, z1, b1.reshape(1, D), W2)
    p2 = _sc_scatter(z2, srcp, dstp, zeroD)
    y = _tc_last_call(degp, p2, z2, b2.reshape(1, D),
                      Wfc.reshape(1, D), bfc.reshape(1, 1))
    return y[:N]


# speedup vs baseline: 1.2995x; 1.0365x over previous
import functools

import jax
import jax.numpy as jnp
from jax import lax
from jax.experimental import pallas as pl
from jax.experimental.pallas import tpu as pltpu
from jax.experimental.pallas import tpu_sc as plsc

N = 10000
D = 128
E = 320000
NC, NS = 2, 16
NW = NC * NS
NPAD = 10240
RB = NPAD // NS
CHUNK = 128
CH = 79
UNROLL = 8
EPT = CH * CHUNK
E_PAD = EPT * NW
TCB = 256
GRID = NPAD // TCB

_mesh = plsc.VectorSubcoreMesh(
    core_axis_name="c", subcore_axis_name="s", num_cores=NC, num_subcores=NS)


@functools.partial(
    pl.kernel,
    out_type=jax.ShapeDtypeStruct((NC, NPAD, D), jnp.float32),
    mesh=_mesh,
    scratch_types=[
        pltpu.VMEM_SHARED((NPAD, D), jnp.float32),
        pltpu.VMEM((CHUNK, D), jnp.float32),
        pltpu.VMEM((CHUNK,), jnp.int32),
        pltpu.VMEM((CHUNK,), jnp.int32),
        pltpu.VMEM((CHUNK,), jnp.int32),
        pltpu.VMEM((CHUNK,), jnp.int32),
        pltpu.SemaphoreType.DMA,
    ],
)
def _sc_deg(dst_hbm, zero_hbm, ones_hbm, out_hbm, acc, ones_v, idx_a, idx_b,
            idx_c, idx_d, sem):
    c = lax.axis_index("c")
    s = lax.axis_index("s")
    wid = c * NS + s
    pltpu.sync_copy(zero_hbm, acc.at[pl.ds(s * RB, RB)])
    pltpu.sync_copy(ones_hbm, ones_v)
    plsc.subcore_barrier()

    idxs = (idx_a, idx_b, idx_c, idx_d)

    def body(g, carry):
        base = wid * EPT + g * (4 * CHUNK)
        descs = []
        for j in range(4):
            pltpu.sync_copy(dst_hbm.at[pl.ds(base + j * CHUNK, CHUNK)],
                            idxs[j])
            descs.append(
                pltpu.async_copy(ones_v, acc.at[idxs[j]], sem, add=True))
        for d in descs:
            d.wait()
        return carry

    lax.fori_loop(0, CH // 4, body, 0)
    for k in range((CH // 4) * 4, CH):
        base = wid * EPT + k * CHUNK
        pltpu.sync_copy(dst_hbm.at[pl.ds(base, CHUNK)], idx_a)
        pltpu.sync_copy(ones_v, acc.at[idx_a], add=True)
    plsc.subcore_barrier()
    pltpu.sync_copy(acc.at[pl.ds(s * RB, RB)], out_hbm.at[c, pl.ds(s * RB, RB)])


@functools.partial(
    pl.kernel,
    out_type=jax.ShapeDtypeStruct((NC, NPAD, D), jnp.float32),
    mesh=_mesh,
    scratch_types=[
        pltpu.VMEM_SHARED((NPAD, D), jnp.float32),
        pltpu.VMEM((CHUNK, D), jnp.float32),
        pltpu.VMEM((CHUNK, D), jnp.float32),
        pltpu.VMEM((CHUNK,), jnp.int32),
        pltpu.VMEM((CHUNK,), jnp.int32),
        pltpu.VMEM((CHUNK,), jnp.int32),
        pltpu.VMEM((CHUNK,), jnp.int32),
        pltpu.SemaphoreType.DMA,
        pltpu.SemaphoreType.DMA,
        pltpu.SemaphoreType.DMA,
    ],
)
def _sc_scatter(z_hbm, src_hbm, dst_hbm, zero_hbm, out_hbm,
                acc, rows_a, rows_b, sidx_a, sidx_b, didx_a, didx_b,
                gsem, ssem, isem):
    c = lax.axis_index("c")
    s = lax.axis_index("s")
    wid = c * NS + s
    pltpu.sync_copy(zero_hbm, acc.at[pl.ds(s * RB, RB)])
    plsc.subcore_barrier()

    def body(g, carry):
        base = wid * EPT + g * (2 * CHUNK)
        ia = pltpu.async_copy(src_hbm.at[pl.ds(base, CHUNK)], sidx_a, isem)
        ib = pltpu.async_copy(dst_hbm.at[pl.ds(base, CHUNK)], didx_a, isem)
        ia.wait()
        ib.wait()
        ga = pltpu.async_copy(z_hbm.at[sidx_a], rows_a, gsem)

        @pl.when(g > 0)
        def _drain():
            pltpu.make_async_copy(rows_b, acc.at[didx_b], ssem).wait()

        ic = pltpu.async_copy(src_hbm.at[pl.ds(base + CHUNK, CHUNK)],
                              sidx_b, isem)
        id_ = pltpu.async_copy(dst_hbm.at[pl.ds(base + CHUNK, CHUNK)],
                               didx_b, isem)
        ic.wait()
        id_.wait()
        ga.wait()
        gb = pltpu.async_copy(z_hbm.at[sidx_b], rows_b, gsem)
        d1 = pltpu.async_copy(rows_a, acc.at[didx_a], ssem, add=True)
        gb.wait()
        pltpu.async_copy(rows_b, acc.at[didx_b], ssem, add=True)
        d1.wait()
        return carry

    lax.fori_loop(0, CH // 2, body, 0)
    pltpu.make_async_copy(rows_b, acc.at[didx_b], ssem).wait()
    base = wid * EPT + (CH - 1) * CHUNK
    pltpu.sync_copy(src_hbm.at[pl.ds(base, CHUNK)], sidx_a)
    pltpu.sync_copy(dst_hbm.at[pl.ds(base, CHUNK)], didx_a)
    pltpu.sync_copy(z_hbm.at[sidx_a], rows_a)
    pltpu.sync_copy(rows_a, acc.at[didx_a], add=True)
    plsc.subcore_barrier()
    pltpu.sync_copy(acc.at[pl.ds(s * RB, RB)], out_hbm.at[c, pl.ds(s * RB, RB)])


def _dinv_block(degp):
    deg = degp[0, :, 0:1] + degp[1, :, 0:1] + 1.0
    return lax.rsqrt(deg)


def _row_mask(i):
    rows = i * TCB + lax.broadcasted_iota(jnp.int32, (TCB, 1), 0)
    return rows < N


def _tc_first(degp_ref, x_ref, w_ref, z_ref):
    i = pl.program_id(0)
    dinv = _dinv_block(degp_ref[...])
    xw = jnp.dot(x_ref[...], w_ref[...], preferred_element_type=jnp.float32)
    z_ref[...] = jnp.where(_row_mask(i), xw * dinv, 0.0)


def _tc_mid(degp_ref, p_ref, z_ref, b_ref, w_ref, z2_ref):
    i = pl.program_id(0)
    dinv = _dinv_block(degp_ref[...])
    ssum = p_ref[0] + p_ref[1] + z_ref[...]
    h = jnp.maximum(dinv * ssum + b_ref[...], 0.0)
    h = jnp.where(_row_mask(i), h, 0.0)
    z2_ref[...] = jnp.dot(h, w_ref[...], preferred_element_type=jnp.float32) * dinv


def _tc_last(degp_ref, p_ref, z_ref, b_ref, wfc_ref, bfc_ref, y_ref):
    i = pl.program_id(0)
    dinv = _dinv_block(degp_ref[...])
    ssum = p_ref[0] + p_ref[1] + z_ref[...]
    h = jnp.maximum(dinv * ssum + b_ref[...], 0.0)
    h = jnp.where(_row_mask(i), h, 0.0)
    y_ref[...] = jnp.sum(h * wfc_ref[...], axis=1, keepdims=True) + bfc_ref[0, 0]


_degp_spec = pl.BlockSpec((NC, TCB, D), lambda i: (0, i, 0))
_rows_spec = pl.BlockSpec((TCB, D), lambda i: (i, 0))
_parts_spec = pl.BlockSpec((NC, TCB, D), lambda i: (0, i, 0))
_w_spec = pl.BlockSpec((D, D), lambda i: (0, 0))
_b_spec = pl.BlockSpec((1, D), lambda i: (0, 0))


def _tc_first_call(degp, x, w):
    return pl.pallas_call(
        _tc_first,
        grid=(GRID,),
        in_specs=[_degp_spec, _rows_spec, _w_spec],
        out_specs=_rows_spec,
        out_shape=jax.ShapeDtypeStruct((NPAD, D), jnp.float32),
    )(degp, x, w)


def _tc_mid_call(degp, parts, z, b2d, w):
    return pl.pallas_call(
        _tc_mid,
        grid=(GRID,),
        in_specs=[_degp_spec, _parts_spec, _rows_spec, _b_spec, _w_spec],
        out_specs=_rows_spec,
        out_shape=jax.ShapeDtypeStruct((NPAD, D), jnp.float32),
    )(degp, parts, z, b2d, w)


def _tc_last_call(degp, parts, z, b2d, wfc_row, bfc2d):
    return pl.pallas_call(
        _tc_last,
        grid=(GRID,),
        in_specs=[_degp_spec, _parts_spec, _rows_spec, _b_spec, _b_spec,
                  pl.BlockSpec((1, 1), lambda i: (0, 0))],
        out_specs=pl.BlockSpec((TCB, 1), lambda i: (i, 0)),
        out_shape=jax.ShapeDtypeStruct((NPAD, 1), jnp.float32),
    )(degp, parts, z, b2d, wfc_row, bfc2d)


def kernel(x, edge_index, W1, b1, W2, b2, Wfc, bfc):
    ei = edge_index.astype(jnp.int32)
    pad = jnp.full((E_PAD - E,), N, jnp.int32)
    srcp = jnp.concatenate([ei[0], pad])
    dstp = jnp.concatenate([ei[1], pad])

    zeroD = jnp.zeros((RB, D), jnp.float32)
    onesD = jnp.ones((CHUNK, D), jnp.float32)

    degp = _sc_deg(dstp, zeroD, onesD)

    z1 = _tc_first_call(degp, x, W1)
    p1 = _sc_scatter(z1, srcp, dstp, zeroD)
    z2 = _tc_mid_call(degp, p1, z1, b1.reshape(1, D), W2)
    p2 = _sc_scatter(z2, srcp, dstp, zeroD)
    y = _tc_last_call(degp, p2, z2, b2.reshape(1, D),
                      Wfc.reshape(1, D), bfc.reshape(1, 1))
    return y[:N]
